# baseline (speedup 1.0000x reference)
import functools

import jax
import jax.numpy as jnp
from jax import lax
from jax.experimental import pallas as pl
from jax.experimental.pallas import tpu as pltpu
from jax.experimental.pallas import tpu_sc as plsc

N_NODES = 10000
N_PAD = 10240
E_TOT = 320000
H = 128
NC, NS = 2, 16
NW = NC * NS
EW = E_TOT // NW
CH = 40
NB = 4
NCHUNK = EW // CH
NOUT = (NCHUNK - 2) // NB
STRIPE = N_PAD // NS


def _gather_body(y_hbm, eap_hbm, src_hbm, z_hbm, *rest):
    src_v = [rest[3 * b + 0] for b in range(NB)]
    ea_v = [rest[3 * b + 1] for b in range(NB)]
    rows_v = [rest[3 * b + 2] for b in range(NB)]
    sem_i = list(rest[3 * NB:4 * NB])
    sem_g = list(rest[4 * NB:5 * NB])
    sem_o = list(rest[5 * NB:6 * NB])

    cid = lax.axis_index("c")
    sid = lax.axis_index("s")
    base_w = (sid * NC + cid) * EW

    def fire_idx(c, b):
        off = base_w + c * CH
        pltpu.async_copy(src_hbm.at[pl.ds(off, CH)], src_v[b], sem_i[b])
        pltpu.async_copy(eap_hbm.at[pl.ds(off, CH)], ea_v[b], sem_i[b])

    def wait_idx(b):
        pltpu.make_async_copy(src_hbm.at[pl.ds(0, CH)], src_v[b], sem_i[b]).wait()
        pltpu.make_async_copy(eap_hbm.at[pl.ds(0, CH)], ea_v[b], sem_i[b]).wait()

    def fire_gather(b):
        pltpu.async_copy(y_hbm.at[src_v[b]], rows_v[b], sem_g[b])

    def wait_gather(b):
        pltpu.make_async_copy(y_hbm.at[src_v[b]], rows_v[b], sem_g[b]).wait()

    def fire_out(c, b):
        off = base_w + c * CH
        pltpu.async_copy(rows_v[b], z_hbm.at[pl.ds(off, CH)], sem_o[b])

    def wait_out(b):
        pltpu.make_async_copy(rows_v[b], z_hbm.at[pl.ds(0, CH)], sem_o[b]).wait()

    def compute(b):
        def ed(e, _):
            for j in range(H // 16):
                r = rows_v[b][e, pl.ds(j * 16, 16)] + ea_v[b][e, pl.ds(j * 16, 16)]
                rows_v[b][e, pl.ds(j * 16, 16)] = jnp.where(r > 0.0, r, 0.2 * r)
            return 0
        lax.fori_loop(0, CH, ed, 0)

    def step(c, b):
        b1 = (b + 1) % NB
        b2 = (b + 2) % NB

        @pl.when(c <= NCHUNK - 3)
        def _():
            fire_idx(c + 2, b2)

        @pl.when(c <= NCHUNK - 2)
        def _():
            wait_idx(b1)

            @pl.when(c >= 3)
            def _():
                wait_out(b1)
            fire_gather(b1)

        wait_gather(b)
        compute(b)
        fire_out(c, b)

    fire_idx(0, 0)
    fire_idx(1, 1)
    wait_idx(0)
    fire_gather(0)

    def outer(g, _):
        for b in range(NB):
            step(g * NB + b, b)
        return 0
    lax.fori_loop(0, NOUT, outer, 0)
    for t in range(NOUT * NB, NCHUNK):
        step(jnp.int32(t), t % NB)

    wait_out((NCHUNK - 4) % NB)
    wait_out((NCHUNK - 3) % NB)
    wait_out((NCHUNK - 2) % NB)
    wait_out((NCHUNK - 1) % NB)


_gather_scratch = []
for _b in range(NB):
    _gather_scratch += [
        pltpu.VMEM((CH,), jnp.int32),
        pltpu.VMEM((CH, H), jnp.float32),
        pltpu.VMEM((CH, H), jnp.float32),
    ]
_gather_scratch += [pltpu.SemaphoreType.DMA] * (3 * NB)

_gather_call = functools.partial(
    pl.kernel,
    out_type=jax.ShapeDtypeStruct((E_TOT, H), jnp.float32),
    mesh=plsc.VectorSubcoreMesh(core_axis_name="c", subcore_axis_name="s"),
    scratch_types=_gather_scratch,
)(_gather_body)


def _scatter_body(h_hbm, dst_hbm, out_hbm, acc, *rest):
    dst_v = [rest[2 * b + 0] for b in range(NB)]
    rows_v = [rest[2 * b + 1] for b in range(NB)]
    sem_i = list(rest[2 * NB:3 * NB])
    sem_s = list(rest[3 * NB:4 * NB])

    cid = lax.axis_index("c")
    sid = lax.axis_index("s")
    base_w = (sid * NC + cid) * EW

    def zb_body(i, _):
        for j in range(H // 16):
            rows_v[0][i, pl.ds(j * 16, 16)] = jnp.zeros((16,), jnp.float32)
        return 0
    lax.fori_loop(0, CH, zb_body, 0)
    for r in range(STRIPE // CH):
        pltpu.sync_copy(rows_v[0], acc.at[pl.ds(sid * STRIPE + r * CH, CH)])

    plsc.subcore_barrier()

    def fire_idx(c, b):
        off = base_w + c * CH
        pltpu.async_copy(dst_hbm.at[pl.ds(off, CH)], dst_v[b], sem_i[b])
        pltpu.async_copy(h_hbm.at[pl.ds(off, CH)], rows_v[b], sem_i[b])

    def wait_idx(b):
        pltpu.make_async_copy(dst_hbm.at[pl.ds(0, CH)], dst_v[b], sem_i[b]).wait()
        pltpu.make_async_copy(h_hbm.at[pl.ds(0, CH)], rows_v[b], sem_i[b]).wait()

    def fire_scatter(b):
        pltpu.async_copy(rows_v[b], acc.at[dst_v[b]], sem_s[b], add=True)

    def wait_scatter(b):
        pltpu.make_async_copy(rows_v[b], acc.at[dst_v[b]], sem_s[b]).wait()

    def step(c, b):
        b2 = (b + 2) % NB

        @pl.when(jnp.logical_and(c >= 2, c <= NCHUNK - 3))
        def _():
            wait_scatter(b2)

        @pl.when(c <= NCHUNK - 3)
        def _():
            fire_idx(c + 2, b2)

        wait_idx(b)
        fire_scatter(b)

    fire_idx(0, 0)
    fire_idx(1, 1)

    def outer(g, _):
        for b in range(NB):
            step(g * NB + b, b)
        return 0
    lax.fori_loop(0, NOUT, outer, 0)
    for t in range(NOUT * NB, NCHUNK):
        step(jnp.int32(t), t % NB)

    for b in range(NB):
        wait_scatter(b)

    plsc.subcore_barrier()
    pltpu.sync_copy(acc.at[pl.ds(sid * STRIPE, STRIPE)],
                    out_hbm.at[cid, pl.ds(sid * STRIPE, STRIPE)])


_scatter_scratch = [pltpu.VMEM_SHARED((N_PAD, H), jnp.float32)]
for _b in range(NB):
    _scatter_scratch += [
        pltpu.VMEM((CH,), jnp.int32),
        pltpu.VMEM((CH, H), jnp.float32),
    ]
_scatter_scratch += [pltpu.SemaphoreType.DMA] * (2 * NB)

_scatter_call = functools.partial(
    pl.kernel,
    out_type=jax.ShapeDtypeStruct((NC, N_PAD, H), jnp.float32),
    mesh=plsc.VectorSubcoreMesh(core_axis_name="c", subcore_axis_name="s"),
    scratch_types=_scatter_scratch,
)(_scatter_body)


def _mm(x, W, b):
    M, K = x.shape
    Ho = W.shape[1]
    BM = 2000

    def body(x_ref, w_ref, b_ref, o_ref):
        o_ref[...] = jnp.dot(x_ref[...], w_ref[...],
                             preferred_element_type=jnp.float32) + b_ref[...]

    return pl.pallas_call(
        body,
        grid=(M // BM,),
        in_specs=[
            pl.BlockSpec((BM, K), lambda i: (i, 0)),
            pl.BlockSpec((K, Ho), lambda i: (0, 0)),
            pl.BlockSpec((1, Ho), lambda i: (0, 0)),
        ],
        out_specs=pl.BlockSpec((BM, Ho), lambda i: (i, 0)),
        out_shape=jax.ShapeDtypeStruct((M, Ho), jnp.float32),
    )(x, W, b)


def _mm2(agg, W, b):
    BM = 2000

    def body(a0_ref, a1_ref, w_ref, b_ref, o_ref):
        s = a0_ref[0] + a1_ref[0]
        o_ref[...] = jnp.dot(s, w_ref[...],
                             preferred_element_type=jnp.float32) + b_ref[...]

    return pl.pallas_call(
        body,
        grid=(N_NODES // BM,),
        in_specs=[
            pl.BlockSpec((1, BM, H), lambda i: (0, i, 0)),
            pl.BlockSpec((1, BM, H), lambda i: (1, i, 0)),
            pl.BlockSpec((H, H), lambda i: (0, 0)),
            pl.BlockSpec((1, H), lambda i: (0, 0)),
        ],
        out_specs=pl.BlockSpec((BM, H), lambda i: (i, 0)),
        out_shape=jax.ShapeDtypeStruct((N_NODES, H), jnp.float32),
    )(agg, agg, W, b)


def _mm_ea(ea, W1e):
    E, K = ea.shape
    BM = 8000

    def body(a_ref, w_ref, o_ref):
        o_ref[...] = jnp.dot(a_ref[...], w_ref[...],
                             preferred_element_type=jnp.float32)

    return pl.pallas_call(
        body,
        grid=(E // BM,),
        in_specs=[
            pl.BlockSpec((BM, K), lambda i: (i, 0)),
            pl.BlockSpec((K, H), lambda i: (0, 0)),
        ],
        out_specs=pl.BlockSpec((BM, H), lambda i: (i, 0)),
        out_shape=jax.ShapeDtypeStruct((E, H), jnp.float32),
    )(ea, W1e)


def _mm_bn(z, W2, b2, rm, inv, g, bb):
    BM = 8000

    def body(z_ref, w_ref, b_ref, rm_ref, iv_ref, g_ref, bb_ref, o_ref):
        h = jnp.dot(z_ref[...], w_ref[...],
                    preferred_element_type=jnp.float32) + b_ref[...]
        o_ref[...] = (h - rm_ref[...]) * iv_ref[...] * g_ref[...] + bb_ref[...]

    return pl.pallas_call(
        body,
        grid=(E_TOT // BM,),
        in_specs=[
            pl.BlockSpec((BM, H), lambda i: (i, 0)),
            pl.BlockSpec((H, H), lambda i: (0, 0)),
            pl.BlockSpec((1, H), lambda i: (0, 0)),
            pl.BlockSpec((1, H), lambda i: (0, 0)),
            pl.BlockSpec((1, H), lambda i: (0, 0)),
            pl.BlockSpec((1, H), lambda i: (0, 0)),
            pl.BlockSpec((1, H), lambda i: (0, 0)),
        ],
        out_specs=pl.BlockSpec((BM, H), lambda i: (i, 0)),
        out_shape=jax.ShapeDtypeStruct((E_TOT, H), jnp.float32),
    )(z, W2, b2, rm, inv, g, bb)


def _classifier(agg, W1, b1, W2p, b2p):
    BM = 2000

    def body(a0_ref, a1_ref, w1_ref, b1_ref, w2_ref, b2_ref, o_ref):
        x = a0_ref[0] + a1_ref[0]
        h = jnp.dot(x, w1_ref[...],
                    preferred_element_type=jnp.float32) + b1_ref[...]
        h = jnp.where(h > 0.0, h, 0.2 * h)
        o_ref[...] = jax.nn.sigmoid(
            jnp.dot(h, w2_ref[...], preferred_element_type=jnp.float32)
            + b2_ref[...])

    return pl.pallas_call(
        body,
        grid=(N_NODES // BM,),
        in_specs=[
            pl.BlockSpec((1, BM, H), lambda i: (0, i, 0)),
            pl.BlockSpec((1, BM, H), lambda i: (1, i, 0)),
            pl.BlockSpec((H, H), lambda i: (0, 0)),
            pl.BlockSpec((1, H), lambda i: (0, 0)),
            pl.BlockSpec((H, 128), lambda i: (0, 0)),
            pl.BlockSpec((1, 128), lambda i: (0, 0)),
        ],
        out_specs=pl.BlockSpec((BM, 128), lambda i: (i, 0)),
        out_shape=jax.ShapeDtypeStruct((N_NODES, 128), jnp.float32),
    )(agg, agg, W1, b1, W2p, b2p)


def _mpl(y, src, dst, ea, pp, tok):
    y = y + 0.0 * tok.reshape(1, 1)
    eap = _mm_ea(ea, pp['W1'][H:])
    z = _gather_call(y, eap, src)
    inv = (1.0 / jnp.sqrt(pp['bn_rv'] + 1e-5)).reshape(1, H)
    h2 = _mm_bn(z, pp['W2'], pp['b2'].reshape(1, H), pp['bn_rm'].reshape(1, H),
                inv, pp['bn_g'].reshape(1, H), pp['bn_b'].reshape(1, H))
    agg = _scatter_call(h2, dst)
    return agg, agg[0, 0, :1]


def kernel(x_var, x_cons, edge_index, edge_attr, rev_edge_index,
           rev_edge_attr, params):
    p = params
    src_vc = edge_index[0].astype(jnp.int32)
    dst_vc = edge_index[1].astype(jnp.int32)
    src_cv = rev_edge_index[0].astype(jnp.int32)
    dst_cv = rev_edge_index[1].astype(jnp.int32)

    hv = _mm(x_var, p['emb_var_W'], p['emb_var_b'].reshape(1, H))
    hc = _mm(x_cons, p['emb_cons_W'], p['emb_cons_b'].reshape(1, H))

    tok = jnp.zeros((1,), jnp.float32)
    first = True
    for lp in p['layers']:
        if first:
            y_vc = _mm(hv, lp['vc']['W1'][:H], lp['vc']['b1'].reshape(1, H))
            y_cv = _mm(hc, lp['cv']['W1'][:H], lp['cv']['b1'].reshape(1, H))
        else:
            y_vc = _mm2(agg_hv, lp['vc']['W1'][:H], lp['vc']['b1'].reshape(1, H))
            y_cv = _mm2(agg_hc, lp['cv']['W1'][:H], lp['cv']['b1'].reshape(1, H))
        agg_hc, tok = _mpl(y_vc, src_vc, dst_vc, edge_attr, lp['vc'], tok)
        agg_hv, tok = _mpl(y_cv, src_cv, dst_cv, rev_edge_attr, lp['cv'], tok)
        first = False

    W2p = jnp.zeros((H, 128), jnp.float32).at[:, 0].set(p['cls_W2'][:, 0])
    b2p = jnp.zeros((1, 128), jnp.float32).at[0, 0].set(p['cls_b2'][0])
    out = _classifier(agg_hv, p['cls_W1'], p['cls_b1'].reshape(1, H), W2p, b2p)
    return out[:, 0]
